# Initial kernel scaffold; baseline (speedup 1.0000x reference)
import jax
import jax.numpy as jnp
from jax.experimental import pallas as pl


def kernel(graph_features, graph_pos, surface_pos, surface_batch, graph_batch, W1, b1, gamma, beta, W2, b2):
    raise NotImplementedError("write your pallas kernel here")


# speedup vs baseline: 1.1487x; 1.1487x over previous
import functools

import jax
import jax.numpy as jnp
from jax import lax
from jax.experimental import pallas as pl
from jax.experimental.pallas import tpu as pltpu
from jax.experimental.pallas import tpu_sc as plsc


def _dot_bf16x3(a, b):
    a_hi = a.astype(jnp.bfloat16)
    a_lo = (a - a_hi.astype(jnp.float32)).astype(jnp.bfloat16)
    b_hi = b.astype(jnp.bfloat16)
    b_lo = (b - b_hi.astype(jnp.float32)).astype(jnp.bfloat16)
    f32 = jnp.float32
    return (jnp.dot(a_hi, b_hi, preferred_element_type=f32)
            + jnp.dot(a_hi, b_lo, preferred_element_type=f32)
            + jnp.dot(a_lo, b_hi, preferred_element_type=f32))


def _nn_h_body(sp_ref, gt_ref, s2_ref, g2_ref, gf_ref, w1_ref, b1_ref,
               idx_ref, h_ref):
    s = sp_ref[...]
    gt = gt_ref[...]
    s2 = s2_ref[...]
    g2 = g2_ref[...]
    sp = jnp.dot(s, gt, preferred_element_type=jnp.float32)
    dist = jnp.sqrt(jnp.maximum(s2 + g2 - 2.0 * sp, 0.0))
    rowmin = jnp.min(dist, axis=1, keepdims=True)
    col = lax.broadcasted_iota(jnp.int32, dist.shape, 1)
    big = jnp.int32(2**30)
    idx_ref[...] = jnp.min(jnp.where(dist == rowmin, col, big), axis=1,
                           keepdims=True)
    h_ref[...] = _dot_bf16x3(gf_ref[...], w1_ref[...]) + b1_ref[...]


def _make_gather(ng, ns, enc):
    info = plsc.get_sparse_core_info()
    nw = info.num_cores * info.num_subcores
    bpw = ns // nw
    ch = min(bpw, 128)
    mesh = plsc.VectorSubcoreMesh(core_axis_name="c", subcore_axis_name="s")

    @functools.partial(
        pl.kernel, mesh=mesh,
        out_type=jax.ShapeDtypeStruct((ns, enc), jnp.float32),
        scratch_types=[
            pltpu.VMEM((bpw,), jnp.int32),
            pltpu.VMEM((ch, enc), jnp.float32),
            pltpu.SemaphoreType.DMA,
        ],
    )
    def gather_k(h_hbm, idx_hbm, out_hbm, idx_v, rows_v, sem):
        wid = lax.axis_index("s") * info.num_cores + lax.axis_index("c")
        base = wid * bpw
        pltpu.sync_copy(idx_hbm.at[pl.ds(base, bpw)], idx_v)
        for c0 in range(0, bpw, ch):
            pltpu.async_copy(h_hbm.at[idx_v.at[pl.ds(c0, ch)]], rows_v,
                             sem).wait()
            pltpu.sync_copy(rows_v, out_hbm.at[pl.ds(base + c0, ch)])

    return gather_k


def _make_mlp_body(n_rows):
    def _mlp_body(x_ref, gamma_ref, beta_ref, w2_ref, b2_ref, o_ref,
                  sum_ref, sq_ref):
        p = pl.program_id(0)
        i = pl.program_id(1)

        @pl.when((p == 0) & (i == 0))
        def _init():
            sum_ref[...] = jnp.zeros_like(sum_ref)
            sq_ref[...] = jnp.zeros_like(sq_ref)

        x = x_ref[...]

        @pl.when(p == 0)
        def _acc():
            sum_ref[...] += jnp.sum(x, axis=0, keepdims=True)
            sq_ref[...] += jnp.sum(x * x, axis=0, keepdims=True)

        @pl.when(p == 1)
        def _fin():
            n = jnp.float32(n_rows)
            mean = sum_ref[...] / n
            var = sq_ref[...] / n - mean * mean
            inv = lax.rsqrt(var + 1e-5)
            h = (x - mean) * (inv * gamma_ref[...]) + beta_ref[...]
            h = jnp.where(h >= 0, h, 0.2 * h)
            o = _dot_bf16x3(h, w2_ref[...]) + b2_ref[...]
            o_ref[...] = jnp.where(o >= 0, o, 0.2 * o)

    return _mlp_body


def kernel(graph_features, graph_pos, surface_pos, surface_batch, graph_batch,
           W1, b1, gamma, beta, W2, b2):
    ng, enc = graph_features.shape
    ns = surface_pos.shape[0]
    out_dim = W2.shape[1]

    bs = 256
    nb = ns // bs
    bh = ng // nb

    sp_pad = jnp.zeros((ns, 8), jnp.float32).at[:, :3].set(surface_pos)
    gt_pad = jnp.zeros((8, ng), jnp.float32).at[:3, :].set(graph_pos.T)
    s2 = jnp.sum(surface_pos * surface_pos, axis=1, keepdims=True)
    g2 = jnp.sum(graph_pos * graph_pos, axis=1)[None, :]

    idx2d, h_all = pl.pallas_call(
        _nn_h_body,
        grid=(nb,),
        in_specs=[
            pl.BlockSpec((bs, 8), lambda i: (i, 0)),
            pl.BlockSpec((8, ng), lambda i: (0, 0)),
            pl.BlockSpec((bs, 1), lambda i: (i, 0)),
            pl.BlockSpec((1, ng), lambda i: (0, 0)),
            pl.BlockSpec((bh, enc), lambda i: (i, 0)),
            pl.BlockSpec((enc, enc), lambda i: (0, 0)),
            pl.BlockSpec((1, enc), lambda i: (0, 0)),
        ],
        out_specs=[
            pl.BlockSpec((bs, 1), lambda i: (i, 0)),
            pl.BlockSpec((bh, enc), lambda i: (i, 0)),
        ],
        out_shape=[
            jax.ShapeDtypeStruct((ns, 1), jnp.int32),
            jax.ShapeDtypeStruct((ng, enc), jnp.float32),
        ],
    )(sp_pad, gt_pad, s2, g2, graph_features, W1, b1.reshape(1, enc))

    nearest = idx2d.reshape(ns)
    hx = _make_gather(ng, ns, enc)(h_all, nearest)

    bsc = 1024
    nbc = ns // bsc
    out = pl.pallas_call(
        _make_mlp_body(ns),
        grid=(2, nbc),
        in_specs=[
            pl.BlockSpec((bsc, enc), lambda p, i: (i, 0)),
            pl.BlockSpec((1, enc), lambda p, i: (0, 0)),
            pl.BlockSpec((1, enc), lambda p, i: (0, 0)),
            pl.BlockSpec((enc, out_dim), lambda p, i: (0, 0)),
            pl.BlockSpec((1, out_dim), lambda p, i: (0, 0)),
        ],
        out_specs=pl.BlockSpec((bsc, out_dim), lambda p, i: (i, 0)),
        out_shape=jax.ShapeDtypeStruct((ns, out_dim), jnp.float32),
        scratch_shapes=[
            pltpu.VMEM((1, enc), jnp.float32),
            pltpu.VMEM((1, enc), jnp.float32),
        ],
    )(hx, gamma.reshape(1, enc), beta.reshape(1, enc), W2,
      b2.reshape(1, out_dim))

    return out
